# Initial kernel scaffold; baseline (speedup 1.0000x reference)
import jax
import jax.numpy as jnp
from jax.experimental import pallas as pl


def kernel(sparse_idx, dense, tables, W):
    raise NotImplementedError("write your pallas kernel here")


# speedup vs baseline: 1.5577x; 1.5577x over previous
import functools

import jax
import jax.numpy as jnp
from jax import lax
from jax.experimental import pallas as pl
from jax.experimental.pallas import tpu as pltpu
from jax.experimental.pallas import tpu_sc as plsc

VOCAB = 100000
EMB = 16
FIELDS = 26
DENSE = 13
B = 4096

NC = 2
NS = 16
NW = NC * NS
BPW = B // NW

_mesh = plsc.VectorSubcoreMesh(core_axis_name="c", subcore_axis_name="s")

_GATHER_DN = lax.GatherDimensionNumbers(
    offset_dims=(), collapsed_slice_dims=(0,), start_index_map=(0,)
)


def _permute(x, idx16):
    return lax.gather(
        x, idx16[:, None], _GATHER_DN, slice_sizes=(1,),
        mode=lax.GatherScatterMode.PROMISE_IN_BOUNDS,
    )


@functools.partial(
    pl.kernel,
    mesh=_mesh,
    out_type=jax.ShapeDtypeStruct((B,), jnp.float32),
    scratch_types=[
        pltpu.VMEM((FIELDS, BPW), jnp.int32),
        pltpu.VMEM((FIELDS, BPW, EMB), jnp.float32),
        pltpu.VMEM((BPW, 16), jnp.float32),
        pltpu.VMEM((FIELDS, EMB), jnp.float32),
        pltpu.VMEM((16,), jnp.float32),
        pltpu.VMEM((BPW,), jnp.float32),
        pltpu.SemaphoreType.DMA,
    ],
    compiler_params=pltpu.CompilerParams(use_tc_tiling_on_sc=False),
)
def _sc_forward(idx_hbm, dense_hbm, tables_hbm, wf_hbm, wd_hbm, out_hbm,
                idx_v, rows_v, dense_v, wf_v, wd_v, out_v, sem):
    wid = lax.axis_index("s") * NC + lax.axis_index("c")
    base = wid * BPW

    pltpu.sync_copy(idx_hbm.at[wid], idx_v)
    pltpu.sync_copy(dense_hbm.at[pl.ds(base, BPW)], dense_v)
    pltpu.sync_copy(wf_hbm, wf_v)
    pltpu.sync_copy(wd_hbm, wd_v)

    for f in range(1, FIELDS):
        off = jnp.full((16,), f * VOCAB, jnp.int32)
        for j in range(BPW // 16):
            sl = pl.ds(j * 16, 16)
            idx_v[f, sl] = idx_v[f, sl] + off

    copies = [
        pltpu.make_async_copy(tables_hbm.at[idx_v.at[f]], rows_v.at[f], sem)
        for f in range(FIELDS)
    ]
    for c in copies:
        c.start()
    for c in copies:
        c.wait()

    wfs = [wf_v[f] for f in range(FIELDS)]
    wdv = wd_v[...]
    lane = lax.iota(jnp.int32, 16)
    perms = [lane ^ sh for sh in (8, 4, 2, 1)]

    for g in range(BPW // 16):
        def row_body(b, out16):
            i = g * 16 + b
            acc = dense_v[i] * wdv
            for f in range(FIELDS):
                acc = acc + rows_v[f, i] * wfs[f]
            for p in perms:
                acc = acc + _permute(acc, p)
            return jnp.where(lane == b, acc, out16)

        out16 = lax.fori_loop(0, 16, row_body, jnp.zeros((16,), jnp.float32))
        out_v[pl.ds(g * 16, 16)] = 1.0 / (1.0 + jnp.exp(-out16))

    pltpu.sync_copy(out_v, out_hbm.at[pl.ds(base, BPW)])


@jax.jit
def kernel(sparse_idx, dense, tables, W):
    idx_prep = (
        sparse_idx.astype(jnp.int32)
        .reshape(NW, BPW, FIELDS)
        .transpose(0, 2, 1)
    )
    dense_pad = jnp.concatenate(
        [dense, jnp.zeros((B, 16 - DENSE), jnp.float32)], axis=1
    )
    wf = W[: FIELDS * EMB, 0].reshape(FIELDS, EMB)
    wd = jnp.concatenate([W[FIELDS * EMB :, 0], jnp.zeros((16 - DENSE,), jnp.float32)])
    out = _sc_forward(idx_prep, dense_pad, tables, wf, wd)
    return out.reshape(B, 1)
